# baseline (speedup 1.0000x reference)
import functools

import jax
import jax.numpy as jnp
from jax import lax
from jax.experimental import pallas as pl
from jax.experimental.pallas import tpu as pltpu
from jax.experimental.pallas import tpu_sc as plsc

B = 16384
EMB = 16
NLAB = 1000
NC, NS = 2, 16
NW = NC * NS
BPW = B // NW
CHUNK = 128
NCH = BPW // CHUNK
BB = 2048


def _sc_gather(instances, predictors, labels, inst_table, pred_table, label_table):
    mesh = plsc.VectorSubcoreMesh(core_axis_name="c", subcore_axis_name="s",
                                  num_cores=NC, num_subcores=NS)
    out_type = (
        jax.ShapeDtypeStruct((B, EMB), jnp.float32),
        jax.ShapeDtypeStruct((B, EMB), jnp.float32),
        jax.ShapeDtypeStruct((B, EMB), jnp.float32),
    )
    scratch = [
        pltpu.VMEM((NCH, CHUNK), jnp.int32),
        pltpu.VMEM((NCH, CHUNK), jnp.int32),
        pltpu.VMEM((NCH, CHUNK), jnp.int32),
        pltpu.VMEM((BPW, EMB), jnp.float32),
        pltpu.VMEM((BPW, EMB), jnp.float32),
        pltpu.VMEM((BPW, EMB), jnp.float32),
        pltpu.SemaphoreType.DMA,
        pltpu.SemaphoreType.DMA,
        pltpu.SemaphoreType.DMA,
    ]

    @functools.partial(pl.kernel, mesh=mesh, out_type=out_type,
                       scratch_types=scratch,
                       compiler_params=pltpu.CompilerParams(
                           use_tc_tiling_on_sc=False))
    def k(inst_idx_h, pred_idx_h, lab_idx_h, inst_t_h, pred_t_h, lab_t_h,
          inst_o, pred_o, lab_o, ii_v, pi_v, li_v, ir_v, pr_v, lr_v,
          s0, s1, s2):
        wid = lax.axis_index("s") * NC + lax.axis_index("c")
        base = wid * BPW
        pltpu.sync_copy(inst_idx_h.at[wid], ii_v)
        pltpu.sync_copy(pred_idx_h.at[wid], pi_v)
        pltpu.sync_copy(lab_idx_h.at[wid], li_v)
        copies = []
        for j in range(NCH):
            d = pl.ds(j * CHUNK, CHUNK)
            copies.append(pltpu.async_copy(inst_t_h.at[ii_v.at[j]], ir_v.at[d], s0))
            copies.append(pltpu.async_copy(pred_t_h.at[pi_v.at[j]], pr_v.at[d], s1))
            copies.append(pltpu.async_copy(lab_t_h.at[li_v.at[j]], lr_v.at[d], s2))
        for c in copies:
            c.wait()
        out = pl.ds(base, BPW)
        pltpu.sync_copy(ir_v, inst_o.at[out])
        pltpu.sync_copy(pr_v, pred_o.at[out])
        pltpu.sync_copy(lr_v, lab_o.at[out])

    ii = instances.reshape(NW, NCH, CHUNK)
    pp = predictors.reshape(NW, NCH, CHUNK)
    ll = labels.reshape(NW, NCH, CHUNK)
    return k(ii, pp, ll, inst_table, pred_table, label_table)


def _tc_body(inst_ref, pred_ref, lab_ref, wm_ref, bm_ref, wqp_ref, wql_ref,
             bq_ref, out_p_ref, out_q_ref):
    x = jnp.dot(inst_ref[...], wm_ref[...],
                preferred_element_type=jnp.float32,
                precision=lax.Precision.HIGHEST) + bm_ref[...]
    out_p_ref[...] = jnp.minimum(x, 0.0) - jnp.log1p(jnp.exp(-jnp.abs(x)))
    q = jnp.dot(pred_ref[...], wqp_ref[...],
                preferred_element_type=jnp.float32,
                precision=lax.Precision.HIGHEST)
    q = q + jnp.dot(lab_ref[...], wql_ref[...],
                    preferred_element_type=jnp.float32,
                    precision=lax.Precision.HIGHEST)
    out_q_ref[...] = q + bq_ref[...]


def _tc_heads(inst_emb, pred_emb, lab_emb, W_m, b_m, W_q, b_q):
    grid = (B // BB,)
    return pl.pallas_call(
        _tc_body,
        grid=grid,
        in_specs=[
            pl.BlockSpec((BB, EMB), lambda i: (i, 0)),
            pl.BlockSpec((BB, EMB), lambda i: (i, 0)),
            pl.BlockSpec((BB, EMB), lambda i: (i, 0)),
            pl.BlockSpec((EMB, NLAB), lambda i: (0, 0)),
            pl.BlockSpec((1, NLAB), lambda i: (0, 0)),
            pl.BlockSpec((EMB, 4), lambda i: (0, 0)),
            pl.BlockSpec((EMB, 4), lambda i: (0, 0)),
            pl.BlockSpec((1, 4), lambda i: (0, 0)),
        ],
        out_specs=[
            pl.BlockSpec((BB, NLAB), lambda i: (i, 0)),
            pl.BlockSpec((BB, 4), lambda i: (i, 0)),
        ],
        out_shape=[
            jax.ShapeDtypeStruct((B, NLAB), jnp.float32),
            jax.ShapeDtypeStruct((B, 4), jnp.float32),
        ],
    )(inst_emb, pred_emb, lab_emb, W_m, b_m.reshape(1, NLAB),
      W_q[:EMB], W_q[EMB:], b_q.reshape(1, 4))


def kernel(instances, predictors, labels, inst_table, pred_table, label_table,
           W_m, b_m, W_q, b_q):
    inst_emb, pred_emb, lab_emb = _sc_gather(
        instances, predictors, labels, inst_table, pred_table, label_table)
    predictions, q_params = _tc_heads(
        inst_emb, pred_emb, lab_emb, W_m, b_m, W_q, b_q)
    return predictions, q_params


# speedup vs baseline: 1.0782x; 1.0782x over previous
import functools

import jax
import jax.numpy as jnp
from jax import lax
from jax.experimental import pallas as pl
from jax.experimental.pallas import tpu as pltpu
from jax.experimental.pallas import tpu_sc as plsc

B = 16384
EMB = 16
NLAB = 1000
NINST = 1000000
NC, NS = 2, 16
NW = NC * NS
BPW = B // NW
CHUNK = 128
NCH = BPW // CHUNK
BB = 2048


def _sc_gather(instances, predictors, labels, inst_table, pred_table, label_table):
    mesh = plsc.VectorSubcoreMesh(core_axis_name="c", subcore_axis_name="s",
                                  num_cores=NC, num_subcores=NS)
    out_type = (
        jax.ShapeDtypeStruct((B, EMB), jnp.float32),
        jax.ShapeDtypeStruct((B, EMB), jnp.float32),
        jax.ShapeDtypeStruct((B, EMB), jnp.float32),
    )
    scratch = [
        pltpu.VMEM((NCH, CHUNK), jnp.int32),
        pltpu.VMEM((NCH, CHUNK), jnp.int32),
        pltpu.VMEM((NCH, CHUNK), jnp.int32),
        pltpu.VMEM((BPW, EMB), jnp.float32),
        pltpu.VMEM((BPW, EMB), jnp.float32),
        pltpu.VMEM((BPW, EMB), jnp.float32),
        pltpu.SemaphoreType.DMA,
        pltpu.SemaphoreType.DMA,
        pltpu.SemaphoreType.DMA,
    ]

    @functools.partial(pl.kernel, mesh=mesh, out_type=out_type,
                       scratch_types=scratch,
                       compiler_params=pltpu.CompilerParams(
                           use_tc_tiling_on_sc=False))
    def k(inst_idx_h, pred_idx_h, lab_idx_h, inst_t_h, pred_t_h, lab_t_h,
          inst_o, pred_o, lab_o, ii_v, pi_v, li_v, ir_v, pr_v, lr_v,
          s0, s1, s2):
        wid = lax.axis_index("s") * NC + lax.axis_index("c")
        base = wid * BPW
        pltpu.sync_copy(inst_idx_h.at[wid], ii_v)
        pltpu.sync_copy(pred_idx_h.at[wid], pi_v)
        pltpu.sync_copy(lab_idx_h.at[wid], li_v)
        copies = []
        for j in range(NCH):
            d = pl.ds(j * CHUNK, CHUNK)
            copies.append(pltpu.async_copy(inst_t_h.at[ii_v.at[j]], ir_v.at[d], s0))
            copies.append(pltpu.async_copy(pred_t_h.at[pi_v.at[j]], pr_v.at[d], s1))
            copies.append(pltpu.async_copy(lab_t_h.at[li_v.at[j]], lr_v.at[d], s2))
        for c in copies:
            c.wait()
        out = pl.ds(base, BPW)
        pltpu.sync_copy(ir_v, inst_o.at[out])
        pltpu.sync_copy(pr_v, pred_o.at[out])
        pltpu.sync_copy(lr_v, lab_o.at[out])

    flat = lax.optimization_barrier(inst_table.reshape(NINST * EMB))
    inst_lin = flat.reshape(NINST, EMB)
    ii = instances.reshape(NW, NCH, CHUNK)
    pp = predictors.reshape(NW, NCH, CHUNK)
    ll = labels.reshape(NW, NCH, CHUNK)
    return k(ii, pp, ll, inst_lin, pred_table, label_table)


def _tc_body(inst_ref, pred_ref, lab_ref, wm_ref, bm_ref, wqp_ref, wql_ref,
             bq_ref, out_p_ref, out_q_ref):
    x = jnp.dot(inst_ref[...], wm_ref[...],
                preferred_element_type=jnp.float32) + bm_ref[...]
    out_p_ref[...] = jnp.minimum(x, 0.0) - jnp.log1p(jnp.exp(-jnp.abs(x)))
    q = jnp.dot(pred_ref[...], wqp_ref[...],
                preferred_element_type=jnp.float32,
                precision=lax.Precision.HIGHEST)
    q = q + jnp.dot(lab_ref[...], wql_ref[...],
                    preferred_element_type=jnp.float32,
                    precision=lax.Precision.HIGHEST)
    out_q_ref[...] = q + bq_ref[...]


def _tc_heads(inst_emb, pred_emb, lab_emb, W_m, b_m, W_q, b_q):
    grid = (B // BB,)
    return pl.pallas_call(
        _tc_body,
        grid=grid,
        in_specs=[
            pl.BlockSpec((BB, EMB), lambda i: (i, 0)),
            pl.BlockSpec((BB, EMB), lambda i: (i, 0)),
            pl.BlockSpec((BB, EMB), lambda i: (i, 0)),
            pl.BlockSpec((EMB, NLAB), lambda i: (0, 0)),
            pl.BlockSpec((1, NLAB), lambda i: (0, 0)),
            pl.BlockSpec((EMB, 4), lambda i: (0, 0)),
            pl.BlockSpec((EMB, 4), lambda i: (0, 0)),
            pl.BlockSpec((1, 4), lambda i: (0, 0)),
        ],
        out_specs=[
            pl.BlockSpec((BB, NLAB), lambda i: (i, 0)),
            pl.BlockSpec((BB, 4), lambda i: (i, 0)),
        ],
        out_shape=[
            jax.ShapeDtypeStruct((B, NLAB), jnp.float32),
            jax.ShapeDtypeStruct((B, 4), jnp.float32),
        ],
    )(inst_emb, pred_emb, lab_emb, W_m, b_m.reshape(1, NLAB),
      W_q[:EMB], W_q[EMB:], b_q.reshape(1, 4))


def kernel(instances, predictors, labels, inst_table, pred_table, label_table,
           W_m, b_m, W_q, b_q):
    inst_emb, pred_emb, lab_emb = _sc_gather(
        instances, predictors, labels, inst_table, pred_table, label_table)
    predictions, q_params = _tc_heads(
        inst_emb, pred_emb, lab_emb, W_m, b_m, W_q, b_q)
    return predictions, q_params


# speedup vs baseline: 1.1839x; 1.0981x over previous
import functools

import jax
import jax.numpy as jnp
from jax import lax
from jax.experimental import pallas as pl
from jax.experimental.pallas import tpu as pltpu
from jax.experimental.pallas import tpu_sc as plsc

B = 16384
EMB = 16
NLAB = 1000
NINST = 1000000
NC, NS = 2, 16
NW = NC * NS
BPW = B // NW
CHUNK = 128
NCH = BPW // CHUNK
BB = 2048


def _sc_gather(instances, predictors, labels, inst_table, pred_table, label_table):
    mesh = plsc.VectorSubcoreMesh(core_axis_name="c", subcore_axis_name="s",
                                  num_cores=NC, num_subcores=NS)
    out_type = (
        jax.ShapeDtypeStruct((B, EMB), jnp.float32),
        jax.ShapeDtypeStruct((B, EMB), jnp.float32),
        jax.ShapeDtypeStruct((B, EMB), jnp.float32),
    )
    scratch = [
        pltpu.VMEM((NCH, CHUNK), jnp.int32),
        pltpu.VMEM((NCH, CHUNK), jnp.int32),
        pltpu.VMEM((NCH, CHUNK), jnp.int32),
        pltpu.VMEM((BPW, EMB), jnp.float32),
        pltpu.VMEM((BPW, EMB), jnp.float32),
        pltpu.VMEM((BPW, EMB), jnp.float32),
        pltpu.SemaphoreType.DMA,
        pltpu.SemaphoreType.DMA,
        pltpu.SemaphoreType.DMA,
    ]

    @functools.partial(pl.kernel, mesh=mesh, out_type=out_type,
                       scratch_types=scratch,
                       compiler_params=pltpu.CompilerParams(
                           use_tc_tiling_on_sc=False))
    def k(inst_idx_h, pred_idx_h, lab_idx_h, inst_t_h, pred_t_h, lab_t_h,
          inst_o, pred_o, lab_o, ii_v, pi_v, li_v, ir_v, pr_v, lr_v,
          s0, s1, s2):
        wid = lax.axis_index("s") * NC + lax.axis_index("c")
        base = wid * BPW
        pltpu.sync_copy(inst_idx_h.at[wid], ii_v)
        pltpu.sync_copy(pred_idx_h.at[wid], pi_v)
        pltpu.sync_copy(lab_idx_h.at[wid], li_v)
        copies = []
        for j in range(NCH):
            d = pl.ds(j * CHUNK, CHUNK)
            copies.append(pltpu.async_copy(inst_t_h.at[ii_v.at[j]], ir_v.at[d], s0))
            copies.append(pltpu.async_copy(pred_t_h.at[pi_v.at[j]], pr_v.at[d], s1))
            copies.append(pltpu.async_copy(lab_t_h.at[li_v.at[j]], lr_v.at[d], s2))
        for c in copies:
            c.wait()
        out = pl.ds(base, BPW)
        pltpu.sync_copy(ir_v, inst_o.at[out])
        pltpu.sync_copy(pr_v, pred_o.at[out])
        pltpu.sync_copy(lr_v, lab_o.at[out])

    ii = instances.reshape(NW, NCH, CHUNK)
    pp = predictors.reshape(NW, NCH, CHUNK)
    ll = labels.reshape(NW, NCH, CHUNK)
    return k(ii, pp, ll, inst_table, pred_table, label_table)


def _tc_body_t(wm_ref, instT_ref, predT_ref, labT_ref, bm_ref, wqp_ref,
               wql_ref, bq_ref, outT_p_ref, outT_q_ref):
    x = lax.dot_general(wm_ref[...], instT_ref[...], (((0,), (0,)), ((), ())),
                        preferred_element_type=jnp.float32) + bm_ref[...]
    outT_p_ref[...] = jnp.minimum(x, 0.0) - jnp.log1p(jnp.exp(-jnp.abs(x)))
    q = lax.dot_general(wqp_ref[...], predT_ref[...], (((0,), (0,)), ((), ())),
                        preferred_element_type=jnp.float32,
                        precision=lax.Precision.HIGHEST)
    q = q + lax.dot_general(wql_ref[...], labT_ref[...], (((0,), (0,)), ((), ())),
                            preferred_element_type=jnp.float32,
                            precision=lax.Precision.HIGHEST)
    outT_q_ref[...] = q + bq_ref[...]


def _tc_heads(instT, predT, labT, W_m, b_m, W_q, b_q):
    grid = (B // BB,)
    predsT, qT = pl.pallas_call(
        _tc_body_t,
        grid=grid,
        in_specs=[
            pl.BlockSpec((EMB, NLAB), lambda i: (0, 0)),
            pl.BlockSpec((EMB, BB), lambda i: (0, i)),
            pl.BlockSpec((EMB, BB), lambda i: (0, i)),
            pl.BlockSpec((EMB, BB), lambda i: (0, i)),
            pl.BlockSpec((NLAB, 1), lambda i: (0, 0)),
            pl.BlockSpec((EMB, 4), lambda i: (0, 0)),
            pl.BlockSpec((EMB, 4), lambda i: (0, 0)),
            pl.BlockSpec((4, 1), lambda i: (0, 0)),
        ],
        out_specs=[
            pl.BlockSpec((NLAB, BB), lambda i: (0, i)),
            pl.BlockSpec((4, BB), lambda i: (0, i)),
        ],
        out_shape=[
            jax.ShapeDtypeStruct((NLAB, B), jnp.float32),
            jax.ShapeDtypeStruct((4, B), jnp.float32),
        ],
    )(W_m, instT, predT, labT, b_m.reshape(NLAB, 1),
      W_q[:EMB], W_q[EMB:], b_q.reshape(4, 1))
    return predsT.T, qT.T


def kernel(instances, predictors, labels, inst_table, pred_table, label_table,
           W_m, b_m, W_q, b_q):
    inst_emb, pred_emb, lab_emb = _sc_gather(
        instances, predictors, labels, inst_table, pred_table, label_table)
    instT, predT, labT = inst_emb.T, pred_emb.T, lab_emb.T
    predictions, q_params = _tc_heads(instT, predT, labT, W_m, b_m, W_q, b_q)
    return predictions, q_params


# speedup vs baseline: 1.9543x; 1.6507x over previous
import functools

import jax
import jax.numpy as jnp
from jax import lax
from jax.experimental import pallas as pl
from jax.experimental.pallas import tpu as pltpu
from jax.experimental.pallas import tpu_sc as plsc

B = 16384
EMB = 16
NLAB = 1000
NINST = 1000000
NC, NS = 2, 16
NW = NC * NS
BPW = B // NW
CHUNK = 128
NCH = BPW // CHUNK
BB = 2048

NTILE = NINST // 128
TPW = 7808 // NW
CCH = 4
NCHT = TPW // CCH
TAILT = NTILE - NW * TPW
TAILR = NINST - NTILE * 128


def _sc_transpose(inst_table):
    mesh = plsc.VectorSubcoreMesh(core_axis_name="c", subcore_axis_name="s",
                                  num_cores=NC, num_subcores=NS)
    scratch = [
        pltpu.VMEM((2, 8, CCH * 128), jnp.float32),
        pltpu.VMEM((2, 8, CCH * 128), jnp.float32),
        pltpu.VMEM((CCH * 128 * EMB,), jnp.float32),
        pltpu.SemaphoreType.DMA,
        pltpu.SemaphoreType.DMA,
    ]

    @functools.partial(pl.kernel, mesh=mesh,
                       out_type=jax.ShapeDtypeStruct((NINST * EMB,),
                                                     jnp.float32),
                       scratch_types=scratch,
                       compiler_params=pltpu.CompilerParams(
                           use_tc_tiling_on_sc=True,
                           needs_layout_passes=False))
    def k(it3_h, out_h, buf0, buf1, rows_v, sem0, sem1):
        wid = lax.axis_index("s") * NC + lax.axis_index("c")
        t0 = wid * TPW
        bufs = (buf0, buf1)
        sems = (sem0, sem1)
        iota16 = lax.iota(jnp.int32, EMB) * EMB

        def fire(g, b):
            start = pl.multiple_of((t0 + g * CCH) * 128, 128)
            src = it3_h.at[:, :, pl.ds(start, CCH * 128)]
            return pltpu.async_copy(src, bufs[b], sems[b])

        def extract(b):
            buf = bufs[b]
            for t in range(CCH):
                for v in range(8):
                    base = t * 128 + v * 16
                    for c in range(EMB):
                        gg, c8 = divmod(c, 8)
                        val = buf[gg, c8, pl.ds(base, 16)]
                        plsc.store_scatter(rows_v, [iota16 + (base * EMB + c)],
                                           val)

        def flush(g):
            dst = out_h.at[pl.ds((t0 + g * CCH) * 128 * EMB, CCH * 128 * EMB)]
            pltpu.sync_copy(rows_v, dst)

        fire(0, 0).wait()

        def pair(p, _):
            g = p * 2
            c1 = fire(g + 1, 1)
            extract(0)
            flush(g)
            c1.wait()

            @pl.when(g + 2 < NCHT)
            def _():
                fire(g + 2, 0).wait()
            extract(1)
            flush(g + 1)
            return ()

        lax.fori_loop(0, NCHT // 2, pair, (), unroll=False)
        g_last = NCHT - 1
        extract(0)
        flush(g_last)

        @pl.when(wid == 0)
        def _():
            k0 = NW * TPW
            src = it3_h.at[:, :, pl.ds(k0 * 128, TAILT * 128)]
            pltpu.async_copy(src, bufs[0].at[:, :, pl.ds(0, TAILT * 128)],
                             sems[0]).wait()
            extract(0)
            pltpu.sync_copy(rows_v,
                            out_h.at[pl.ds(k0 * 128 * EMB, TAILT * 128 * EMB)])
    it3 = inst_table.T.reshape(2, 8, NINST)
    return k(it3)


def _sc_gather(instances, predictors, labels, flat_table, pred_table,
               label_table):
    mesh = plsc.VectorSubcoreMesh(core_axis_name="c", subcore_axis_name="s",
                                  num_cores=NC, num_subcores=NS)
    out_type = (
        jax.ShapeDtypeStruct((B * EMB,), jnp.float32),
        jax.ShapeDtypeStruct((B, EMB), jnp.float32),
        jax.ShapeDtypeStruct((B, EMB), jnp.float32),
    )
    EPW = BPW * EMB
    NEC = EPW // CHUNK
    scratch = [
        pltpu.VMEM((TAILR * EMB,), jnp.float32),
        pltpu.VMEM((NEC, CHUNK), jnp.int32),
        pltpu.VMEM((NCH, CHUNK), jnp.int32),
        pltpu.VMEM((NCH, CHUNK), jnp.int32),
        pltpu.VMEM((EPW,), jnp.float32),
        pltpu.VMEM((BPW, EMB), jnp.float32),
        pltpu.VMEM((BPW, EMB), jnp.float32),
        pltpu.SemaphoreType.DMA,
        pltpu.SemaphoreType.DMA,
        pltpu.SemaphoreType.DMA,
    ]

    @functools.partial(pl.kernel, mesh=mesh, out_type=out_type,
                       scratch_types=scratch,
                       compiler_params=pltpu.CompilerParams(
                           use_tc_tiling_on_sc=False,
                           needs_layout_passes=False))
    def k(ei_h, pi_h, li_h, ft_h, tt_h, pt_h, lt_h, io_h, po_h, lo_h,
          tail_v, ei_v, pi_v, li_v, ie_v, pr_v, lr_v, s0, s1, s2):
        wid = lax.axis_index("s") * NC + lax.axis_index("c")
        pltpu.sync_copy(tt_h, tail_v)
        pltpu.sync_copy(ei_h.at[wid], ei_v)
        pltpu.sync_copy(pi_h.at[wid], pi_v)
        pltpu.sync_copy(li_h.at[wid], li_v)
        copies = []
        for j in range(NEC):
            copies.append(pltpu.async_copy(
                ft_h.at[ei_v.at[j]], ie_v.at[pl.ds(j * CHUNK, CHUNK)], s0))
        for j in range(NCH):
            d = pl.ds(j * CHUNK, CHUNK)
            copies.append(pltpu.async_copy(pt_h.at[pi_v.at[j]], pr_v.at[d], s1))
            copies.append(pltpu.async_copy(lt_h.at[li_v.at[j]], lr_v.at[d], s2))
        for c in copies:
            c.wait()

        thresh = NTILE * 128 * EMB

        def fixup(j, _):
            for v in range(CHUNK // EMB):
                d = pl.ds(j * CHUNK + v * EMB, EMB)
                e = ei_v.at[j][pl.ds(v * EMB, EMB)]
                m = e >= thresh
                tl = jnp.clip(e - thresh, 0, TAILR * EMB - 1)
                tv = plsc.load_gather(tail_v, [tl])
                ie_v[d] = jnp.where(m, tv, ie_v[d])
            return ()

        lax.fori_loop(0, NEC, fixup, (), unroll=False)
        pltpu.sync_copy(ie_v, io_h.at[pl.ds(wid * EPW, EPW)])
        out = pl.ds(wid * BPW, BPW)
        pltpu.sync_copy(pr_v, po_h.at[out])
        pltpu.sync_copy(lr_v, lo_h.at[out])

    eidx = (instances[:, None] * EMB +
            jnp.arange(EMB, dtype=jnp.int32)).reshape(NW, NEC, CHUNK)
    pp = predictors.reshape(NW, NCH, CHUNK)
    ll = labels.reshape(NW, NCH, CHUNK)
    tail = flat_table[NTILE * 128 * EMB:]
    flat, pred_emb, lab_emb = k(eidx, pp, ll, flat_table, tail, pred_table,
                                label_table)
    return flat.reshape(B, EMB), pred_emb, lab_emb


def _tc_body_t(wm_ref, instT_ref, predT_ref, labT_ref, bm_ref, wqp_ref,
               wql_ref, bq_ref, outT_p_ref, outT_q_ref):
    x = lax.dot_general(wm_ref[...], instT_ref[...], (((0,), (0,)), ((), ())),
                        preferred_element_type=jnp.float32) + bm_ref[...]
    outT_p_ref[...] = jnp.minimum(x, 0.0) - jnp.log1p(jnp.exp(-jnp.abs(x)))
    q = lax.dot_general(wqp_ref[...], predT_ref[...], (((0,), (0,)), ((), ())),
                        preferred_element_type=jnp.float32,
                        precision=lax.Precision.HIGHEST)
    q = q + lax.dot_general(wql_ref[...], labT_ref[...], (((0,), (0,)), ((), ())),
                            preferred_element_type=jnp.float32,
                            precision=lax.Precision.HIGHEST)
    outT_q_ref[...] = q + bq_ref[...]


def _tc_heads(instT, predT, labT, W_m, b_m, W_q, b_q):
    grid = (B // BB,)
    predsT, qT = pl.pallas_call(
        _tc_body_t,
        grid=grid,
        in_specs=[
            pl.BlockSpec((EMB, NLAB), lambda i: (0, 0)),
            pl.BlockSpec((EMB, BB), lambda i: (0, i)),
            pl.BlockSpec((EMB, BB), lambda i: (0, i)),
            pl.BlockSpec((EMB, BB), lambda i: (0, i)),
            pl.BlockSpec((NLAB, 1), lambda i: (0, 0)),
            pl.BlockSpec((EMB, 4), lambda i: (0, 0)),
            pl.BlockSpec((EMB, 4), lambda i: (0, 0)),
            pl.BlockSpec((4, 1), lambda i: (0, 0)),
        ],
        out_specs=[
            pl.BlockSpec((NLAB, BB), lambda i: (0, i)),
            pl.BlockSpec((4, BB), lambda i: (0, i)),
        ],
        out_shape=[
            jax.ShapeDtypeStruct((NLAB, B), jnp.float32),
            jax.ShapeDtypeStruct((4, B), jnp.float32),
        ],
    )(W_m, instT, predT, labT, b_m.reshape(NLAB, 1),
      W_q[:EMB], W_q[EMB:], b_q.reshape(4, 1))
    return predsT.T, qT.T


def kernel(instances, predictors, labels, inst_table, pred_table, label_table,
           W_m, b_m, W_q, b_q):
    flat_table = _sc_transpose(inst_table)
    inst_emb, pred_emb, lab_emb = _sc_gather(
        instances, predictors, labels, flat_table, pred_table, label_table)
    instT, predT, labT = inst_emb.T, pred_emb.T, lab_emb.T
    predictions, q_params = _tc_heads(instT, predT, labT, W_m, b_m, W_q, b_q)
    return predictions, q_params


# speedup vs baseline: 2.4999x; 1.2792x over previous
import functools

import jax
import jax.numpy as jnp
from jax import lax
from jax.experimental import pallas as pl
from jax.experimental.pallas import tpu as pltpu
from jax.experimental.pallas import tpu_sc as plsc

B = 16384
EMB = 16
NLAB = 1000
NINST = 1000000
NC, NS = 2, 16
NW = NC * NS
BPW = B // NW
CHUNK = 128
NCH = BPW // CHUNK
BB = 2048

NTILE = NINST // 128
TPW = 7808 // NW
CCH = 4
NCHT = TPW // CCH
TAILT = NTILE - NW * TPW
TAILR = NINST - NTILE * 128


def _sc_transpose(inst_table):
    mesh = plsc.VectorSubcoreMesh(core_axis_name="c", subcore_axis_name="s",
                                  num_cores=NC, num_subcores=NS)
    scratch = [
        pltpu.VMEM((2, 8, CCH * 128), jnp.float32),
        pltpu.VMEM((2, 8, CCH * 128), jnp.float32),
        pltpu.VMEM((CCH * 128 * EMB,), jnp.float32),
        pltpu.SemaphoreType.DMA,
        pltpu.SemaphoreType.DMA,
    ]

    @functools.partial(pl.kernel, mesh=mesh,
                       out_type=jax.ShapeDtypeStruct((NINST * EMB,),
                                                     jnp.float32),
                       scratch_types=scratch,
                       compiler_params=pltpu.CompilerParams(
                           use_tc_tiling_on_sc=True,
                           needs_layout_passes=False))
    def k(it3_h, out_h, buf0, buf1, rows_v, sem0, sem1):
        wid = lax.axis_index("s") * NC + lax.axis_index("c")
        t0 = wid * TPW
        bufs = (buf0, buf1)
        sems = (sem0, sem1)
        iota16 = lax.iota(jnp.int32, EMB) * EMB

        def fire(g, b):
            start = pl.multiple_of((t0 + g * CCH) * 128, 128)
            src = it3_h.at[:, :, pl.ds(start, CCH * 128)]
            return pltpu.async_copy(src, bufs[b], sems[b])

        def extract(b):
            buf = bufs[b]

            def tile_body(t, _):
                for v in range(8):
                    base = t * 128 + v * 16
                    for c in range(EMB):
                        gg, c8 = divmod(c, 8)
                        val = buf[gg, c8, pl.ds(base, 16)]
                        plsc.store_scatter(rows_v,
                                           [iota16 + (base * EMB + c)], val)
                return ()

            lax.fori_loop(0, CCH, tile_body, (), unroll=False)

        def flush(g):
            dst = out_h.at[pl.ds((t0 + g * CCH) * 128 * EMB, CCH * 128 * EMB)]
            pltpu.sync_copy(rows_v, dst)

        fire(0, 0).wait()

        def pair(p, _):
            g = p * 2
            c1 = fire(g + 1, 1)
            extract(0)
            flush(g)
            c1.wait()
            c2 = fire(g + 2, 0)
            extract(1)
            flush(g + 1)
            c2.wait()
            return ()

        lax.fori_loop(0, (NCHT - 3) // 2, pair, (), unroll=False)
        c1 = fire(NCHT - 2, 1)
        extract(0)
        flush(NCHT - 3)
        c1.wait()
        c2 = fire(NCHT - 1, 0)
        extract(1)
        flush(NCHT - 2)
        c2.wait()
        extract(0)
        flush(NCHT - 1)

        @pl.when(wid == 0)
        def _():
            k0 = NW * TPW
            src = it3_h.at[:, :, pl.ds(k0 * 128, TAILT * 128)]
            pltpu.async_copy(src, bufs[0].at[:, :, pl.ds(0, TAILT * 128)],
                             sems[0]).wait()
            extract(0)
            pltpu.sync_copy(rows_v,
                            out_h.at[pl.ds(k0 * 128 * EMB, TAILT * 128 * EMB)])
    it3 = inst_table.T.reshape(2, 8, NINST)
    return k(it3)


def _sc_gather(instances, predictors, labels, flat_table, pred_table,
               label_table):
    mesh = plsc.VectorSubcoreMesh(core_axis_name="c", subcore_axis_name="s",
                                  num_cores=NC, num_subcores=NS)
    out_type = (
        jax.ShapeDtypeStruct((B * EMB,), jnp.float32),
        jax.ShapeDtypeStruct((B, EMB), jnp.float32),
        jax.ShapeDtypeStruct((B, EMB), jnp.float32),
    )
    EPW = BPW * EMB
    NEC = EPW // CHUNK
    scratch = [
        pltpu.VMEM((TAILR * EMB,), jnp.float32),
        pltpu.VMEM((NEC, CHUNK), jnp.int32),
        pltpu.VMEM((NCH, CHUNK), jnp.int32),
        pltpu.VMEM((NCH, CHUNK), jnp.int32),
        pltpu.VMEM((EPW,), jnp.float32),
        pltpu.VMEM((BPW, EMB), jnp.float32),
        pltpu.VMEM((BPW, EMB), jnp.float32),
        pltpu.SemaphoreType.DMA,
        pltpu.SemaphoreType.DMA,
        pltpu.SemaphoreType.DMA,
    ]

    @functools.partial(pl.kernel, mesh=mesh, out_type=out_type,
                       scratch_types=scratch,
                       compiler_params=pltpu.CompilerParams(
                           use_tc_tiling_on_sc=False,
                           needs_layout_passes=False))
    def k(ei_h, pi_h, li_h, ft_h, tt_h, pt_h, lt_h, io_h, po_h, lo_h,
          tail_v, ei_v, pi_v, li_v, ie_v, pr_v, lr_v, s0, s1, s2):
        wid = lax.axis_index("s") * NC + lax.axis_index("c")
        pltpu.sync_copy(tt_h, tail_v)
        pltpu.sync_copy(ei_h.at[wid], ei_v)
        pltpu.sync_copy(pi_h.at[wid], pi_v)
        pltpu.sync_copy(li_h.at[wid], li_v)
        copies = []
        for j in range(NEC):
            copies.append(pltpu.async_copy(
                ft_h.at[ei_v.at[j]], ie_v.at[pl.ds(j * CHUNK, CHUNK)], s0))
        for j in range(NCH):
            d = pl.ds(j * CHUNK, CHUNK)
            copies.append(pltpu.async_copy(pt_h.at[pi_v.at[j]], pr_v.at[d], s1))
            copies.append(pltpu.async_copy(lt_h.at[li_v.at[j]], lr_v.at[d], s2))
        for c in copies:
            c.wait()

        thresh = NTILE * 128 * EMB

        def fixup(j, _):
            for v in range(CHUNK // EMB):
                d = pl.ds(j * CHUNK + v * EMB, EMB)
                e = ei_v.at[j][pl.ds(v * EMB, EMB)]
                m = e >= thresh
                tl = jnp.clip(e - thresh, 0, TAILR * EMB - 1)
                tv = plsc.load_gather(tail_v, [tl])
                ie_v[d] = jnp.where(m, tv, ie_v[d])
            return ()

        lax.fori_loop(0, NEC, fixup, (), unroll=False)
        pltpu.sync_copy(ie_v, io_h.at[pl.ds(wid * EPW, EPW)])
        out = pl.ds(wid * BPW, BPW)
        pltpu.sync_copy(pr_v, po_h.at[out])
        pltpu.sync_copy(lr_v, lo_h.at[out])

    eidx = (instances[:, None] * EMB +
            jnp.arange(EMB, dtype=jnp.int32)).reshape(NW, NEC, CHUNK)
    pp = predictors.reshape(NW, NCH, CHUNK)
    ll = labels.reshape(NW, NCH, CHUNK)
    tail = flat_table[NTILE * 128 * EMB:]
    flat, pred_emb, lab_emb = k(eidx, pp, ll, flat_table, tail, pred_table,
                                label_table)
    return flat.reshape(B, EMB), pred_emb, lab_emb


def _tc_body_t(wm_ref, instT_ref, predT_ref, labT_ref, bm_ref, wqp_ref,
               wql_ref, bq_ref, outT_p_ref, outT_q_ref):
    x = lax.dot_general(wm_ref[...], instT_ref[...], (((0,), (0,)), ((), ())),
                        preferred_element_type=jnp.float32) + bm_ref[...]
    outT_p_ref[...] = jnp.minimum(x, 0.0) - jnp.log1p(jnp.exp(-jnp.abs(x)))
    q = lax.dot_general(wqp_ref[...], predT_ref[...], (((0,), (0,)), ((), ())),
                        preferred_element_type=jnp.float32,
                        precision=lax.Precision.HIGHEST)
    q = q + lax.dot_general(wql_ref[...], labT_ref[...], (((0,), (0,)), ((), ())),
                            preferred_element_type=jnp.float32,
                            precision=lax.Precision.HIGHEST)
    outT_q_ref[...] = q + bq_ref[...]


def _tc_heads(instT, predT, labT, W_m, b_m, W_q, b_q):
    grid = (B // BB,)
    predsT, qT = pl.pallas_call(
        _tc_body_t,
        grid=grid,
        in_specs=[
            pl.BlockSpec((EMB, NLAB), lambda i: (0, 0)),
            pl.BlockSpec((EMB, BB), lambda i: (0, i)),
            pl.BlockSpec((EMB, BB), lambda i: (0, i)),
            pl.BlockSpec((EMB, BB), lambda i: (0, i)),
            pl.BlockSpec((NLAB, 1), lambda i: (0, 0)),
            pl.BlockSpec((EMB, 4), lambda i: (0, 0)),
            pl.BlockSpec((EMB, 4), lambda i: (0, 0)),
            pl.BlockSpec((4, 1), lambda i: (0, 0)),
        ],
        out_specs=[
            pl.BlockSpec((NLAB, BB), lambda i: (0, i)),
            pl.BlockSpec((4, BB), lambda i: (0, i)),
        ],
        out_shape=[
            jax.ShapeDtypeStruct((NLAB, B), jnp.float32),
            jax.ShapeDtypeStruct((4, B), jnp.float32),
        ],
    )(W_m, instT, predT, labT, b_m.reshape(NLAB, 1),
      W_q[:EMB], W_q[EMB:], b_q.reshape(4, 1))
    return predsT.T, qT.T


def kernel(instances, predictors, labels, inst_table, pred_table, label_table,
           W_m, b_m, W_q, b_q):
    flat_table = _sc_transpose(inst_table)
    inst_emb, pred_emb, lab_emb = _sc_gather(
        instances, predictors, labels, flat_table, pred_table, label_table)
    instT, predT, labT = inst_emb.T, pred_emb.T, lab_emb.T
    predictions, q_params = _tc_heads(instT, predT, labT, W_m, b_m, W_q, b_q)
    return predictions, q_params


# speedup vs baseline: 2.5016x; 1.0007x over previous
import functools

import jax
import jax.numpy as jnp
from jax import lax
from jax.experimental import pallas as pl
from jax.experimental.pallas import tpu as pltpu
from jax.experimental.pallas import tpu_sc as plsc

B = 16384
EMB = 16
NLAB = 1000
NINST = 1000000
NC, NS = 2, 16
NW = NC * NS
BPW = B // NW
CHUNK = 128
NCH = BPW // CHUNK
BB = 2048

NTILE = NINST // 128
TPW = 7808 // NW
CCH = 4
NCHT = TPW // CCH
TAILT = NTILE - NW * TPW
TAILR = NINST - NTILE * 128


def _sc_transpose(inst_table):
    mesh = plsc.VectorSubcoreMesh(core_axis_name="c", subcore_axis_name="s",
                                  num_cores=NC, num_subcores=NS)
    scratch = [
        pltpu.VMEM((2, 8, CCH * 128), jnp.float32),
        pltpu.VMEM((2, 8, CCH * 128), jnp.float32),
        pltpu.VMEM((CCH * 128 * EMB,), jnp.float32),
        pltpu.SemaphoreType.DMA,
        pltpu.SemaphoreType.DMA,
    ]

    @functools.partial(pl.kernel, mesh=mesh,
                       out_type=jax.ShapeDtypeStruct((NINST * EMB,),
                                                     jnp.float32),
                       scratch_types=scratch,
                       compiler_params=pltpu.CompilerParams(
                           use_tc_tiling_on_sc=True,
                           needs_layout_passes=False))
    def k(it3_h, out_h, buf0, buf1, rows_v, sem0, sem1):
        wid = lax.axis_index("s") * NC + lax.axis_index("c")
        t0 = wid * TPW
        bufs = (buf0, buf1)
        sems = (sem0, sem1)
        iota16 = lax.iota(jnp.int32, EMB) * EMB

        def fire(g, b):
            start = pl.multiple_of((t0 + g * CCH) * 128, 128)
            src = it3_h.at[:, :, pl.ds(start, CCH * 128)]
            return pltpu.async_copy(src, bufs[b], sems[b])

        def extract(b):
            buf = bufs[b]

            def tile_body(t, _):
                for v in range(8):
                    base = t * 128 + v * 16
                    for c in range(EMB):
                        gg, c8 = divmod(c, 8)
                        val = buf[gg, c8, pl.ds(base, 16)]
                        plsc.store_scatter(rows_v,
                                           [iota16 + (base * EMB + c)], val)
                return ()

            lax.fori_loop(0, CCH, tile_body, (), unroll=False)

        def flush(g):
            dst = out_h.at[pl.ds((t0 + g * CCH) * 128 * EMB, CCH * 128 * EMB)]
            pltpu.sync_copy(rows_v, dst)

        fire(0, 0).wait()

        def pair(p, _):
            g = p * 2
            c1 = fire(g + 1, 1)
            extract(0)
            flush(g)
            c1.wait()
            c2 = fire(g + 2, 0)
            extract(1)
            flush(g + 1)
            c2.wait()
            return ()

        lax.fori_loop(0, (NCHT - 3) // 2, pair, (), unroll=False)
        c1 = fire(NCHT - 2, 1)
        extract(0)
        flush(NCHT - 3)
        c1.wait()
        c2 = fire(NCHT - 1, 0)
        extract(1)
        flush(NCHT - 2)
        c2.wait()
        extract(0)
        flush(NCHT - 1)

        @pl.when(wid == 0)
        def _():
            k0 = NW * TPW
            src = it3_h.at[:, :, pl.ds(k0 * 128, TAILT * 128)]
            pltpu.async_copy(src, bufs[0].at[:, :, pl.ds(0, TAILT * 128)],
                             sems[0]).wait()
            extract(0)
            pltpu.sync_copy(rows_v,
                            out_h.at[pl.ds(k0 * 128 * EMB, TAILT * 128 * EMB)])
    it3 = inst_table.T.reshape(2, 8, NINST)
    return k(it3)


def _sc_gather(instances, predictors, labels, flat_table, inst_table_tail,
               pred_table, label_table):
    mesh = plsc.VectorSubcoreMesh(core_axis_name="c", subcore_axis_name="s",
                                  num_cores=NC, num_subcores=NS)
    out_type = (
        jax.ShapeDtypeStruct((B * EMB,), jnp.float32),
        jax.ShapeDtypeStruct((B, EMB), jnp.float32),
        jax.ShapeDtypeStruct((B, EMB), jnp.float32),
    )
    EPW = BPW * EMB
    NEC = EPW // CHUNK
    scratch = [
        pltpu.VMEM((TAILR * EMB,), jnp.float32),
        pltpu.VMEM((NEC, CHUNK), jnp.int32),
        pltpu.VMEM((NCH, CHUNK), jnp.int32),
        pltpu.VMEM((NCH, CHUNK), jnp.int32),
        pltpu.VMEM((EPW,), jnp.float32),
        pltpu.VMEM((BPW, EMB), jnp.float32),
        pltpu.VMEM((BPW, EMB), jnp.float32),
        pltpu.SemaphoreType.DMA,
        pltpu.SemaphoreType.DMA,
        pltpu.SemaphoreType.DMA,
    ]

    @functools.partial(pl.kernel, mesh=mesh, out_type=out_type,
                       scratch_types=scratch,
                       compiler_params=pltpu.CompilerParams(
                           use_tc_tiling_on_sc=False,
                           needs_layout_passes=False))
    def k(ei_h, pi_h, li_h, ft_h, tt_h, pt_h, lt_h, io_h, po_h, lo_h,
          tail_v, ei_v, pi_v, li_v, ie_v, pr_v, lr_v, s0, s1, s2):
        wid = lax.axis_index("s") * NC + lax.axis_index("c")
        pltpu.sync_copy(tt_h, tail_v)
        pltpu.sync_copy(ei_h.at[wid], ei_v)
        pltpu.sync_copy(pi_h.at[wid], pi_v)
        pltpu.sync_copy(li_h.at[wid], li_v)
        copies = []
        for j in range(NEC):
            copies.append(pltpu.async_copy(
                ft_h.at[ei_v.at[j]], ie_v.at[pl.ds(j * CHUNK, CHUNK)], s0))
        for j in range(NCH):
            d = pl.ds(j * CHUNK, CHUNK)
            copies.append(pltpu.async_copy(pt_h.at[pi_v.at[j]], pr_v.at[d], s1))
            copies.append(pltpu.async_copy(lt_h.at[li_v.at[j]], lr_v.at[d], s2))
        for c in copies:
            c.wait()

        thresh = NTILE * 128 * EMB

        def fixup(j, _):
            for v in range(CHUNK // EMB):
                d = pl.ds(j * CHUNK + v * EMB, EMB)
                e = ei_v.at[j][pl.ds(v * EMB, EMB)]
                m = e >= thresh
                tl = jnp.clip(e - thresh, 0, TAILR * EMB - 1)
                tv = plsc.load_gather(tail_v, [tl])
                ie_v[d] = jnp.where(m, tv, ie_v[d])
            return ()

        lax.fori_loop(0, NEC, fixup, (), unroll=False)
        pltpu.sync_copy(ie_v, io_h.at[pl.ds(wid * EPW, EPW)])
        out = pl.ds(wid * BPW, BPW)
        pltpu.sync_copy(pr_v, po_h.at[out])
        pltpu.sync_copy(lr_v, lo_h.at[out])

    eidx = (instances[:, None] * EMB +
            jnp.arange(EMB, dtype=jnp.int32)).reshape(NW, NEC, CHUNK)
    pp = predictors.reshape(NW, NCH, CHUNK)
    ll = labels.reshape(NW, NCH, CHUNK)
    tail = inst_table_tail.reshape(TAILR * EMB)
    flat, pred_emb, lab_emb = k(eidx, pp, ll, flat_table, tail, pred_table,
                                label_table)
    return flat.reshape(B, EMB), pred_emb, lab_emb


def _tc_body_t(wm_ref, instT_ref, predT_ref, labT_ref, bm_ref, wqp_ref,
               wql_ref, bq_ref, outT_p_ref, outT_q_ref):
    x = lax.dot_general(wm_ref[...], instT_ref[...], (((0,), (0,)), ((), ())),
                        preferred_element_type=jnp.float32) + bm_ref[...]
    outT_p_ref[...] = jnp.minimum(x, 0.0) - jnp.log1p(jnp.exp(-jnp.abs(x)))
    q = lax.dot_general(wqp_ref[...], predT_ref[...], (((0,), (0,)), ((), ())),
                        preferred_element_type=jnp.float32,
                        precision=lax.Precision.HIGHEST)
    q = q + lax.dot_general(wql_ref[...], labT_ref[...], (((0,), (0,)), ((), ())),
                            preferred_element_type=jnp.float32,
                            precision=lax.Precision.HIGHEST)
    outT_q_ref[...] = q + bq_ref[...]


def _tc_heads(instT, predT, labT, W_m, b_m, W_q, b_q):
    grid = (B // BB,)
    predsT, qT = pl.pallas_call(
        _tc_body_t,
        grid=grid,
        in_specs=[
            pl.BlockSpec((EMB, NLAB), lambda i: (0, 0)),
            pl.BlockSpec((EMB, BB), lambda i: (0, i)),
            pl.BlockSpec((EMB, BB), lambda i: (0, i)),
            pl.BlockSpec((EMB, BB), lambda i: (0, i)),
            pl.BlockSpec((NLAB, 1), lambda i: (0, 0)),
            pl.BlockSpec((EMB, 4), lambda i: (0, 0)),
            pl.BlockSpec((EMB, 4), lambda i: (0, 0)),
            pl.BlockSpec((4, 1), lambda i: (0, 0)),
        ],
        out_specs=[
            pl.BlockSpec((NLAB, BB), lambda i: (0, i)),
            pl.BlockSpec((4, BB), lambda i: (0, i)),
        ],
        out_shape=[
            jax.ShapeDtypeStruct((NLAB, B), jnp.float32),
            jax.ShapeDtypeStruct((4, B), jnp.float32),
        ],
    )(W_m, instT, predT, labT, b_m.reshape(NLAB, 1),
      W_q[:EMB], W_q[EMB:], b_q.reshape(4, 1))
    return predsT.T, qT.T


def kernel(instances, predictors, labels, inst_table, pred_table, label_table,
           W_m, b_m, W_q, b_q):
    flat_table = _sc_transpose(inst_table)
    inst_emb, pred_emb, lab_emb = _sc_gather(
        instances, predictors, labels, flat_table, inst_table[NTILE * 128:],
        pred_table, label_table)
    instT, predT, labT = inst_emb.T, pred_emb.T, lab_emb.T
    predictions, q_params = _tc_heads(instT, predT, labT, W_m, b_m, W_q, b_q)
    return predictions, q_params
